# baseline (speedup 1.0000x reference)
import math

import numpy as np
import jax
import jax.numpy as jnp
from jax import lax
from jax.experimental import pallas as pl
from jax.experimental.pallas import tpu as pltpu
from jax.experimental.pallas import tpu_sc as plsc

D_MODEL = 128
SEQ_LEN = 200
SCALE = math.sqrt(float(D_MODEL))
CHUNK = 128
NUM_WORKERS = 32
TOTAL_ROWS = 4096 * SEQ_LEN
CHUNKS_TOTAL = TOTAL_ROWS // CHUNK
CHUNKS_PER_WORKER = CHUNKS_TOTAL // NUM_WORKERS
PE_ROWS = 2 * SEQ_LEN
LANES = 16


def _make_pe_const() -> np.ndarray:
    position = np.arange(0, SEQ_LEN, dtype=np.float32)[:, None]
    div_term = np.exp(
        np.arange(0, D_MODEL, 2, dtype=np.float32) * -(math.log(10000.0) / D_MODEL)
    )
    pe = np.zeros((SEQ_LEN, D_MODEL), dtype=np.float32)
    pe[:, 0::2] = np.sin(position * div_term)
    pe[:, 1::2] = np.cos(position * div_term)
    return np.concatenate([pe, pe], axis=0)


_PE2 = _make_pe_const()


def _sc_body(lut_hbm, idx_hbm, pe_hbm, out_hbm, idx_v, pe_v, rows_v, sem):
    core = lax.axis_index("c")
    sub = lax.axis_index("s")
    wid = sub * 2 + core
    chunk_base = wid * CHUNKS_PER_WORKER

    pltpu.sync_copy(pe_hbm, pe_v)
    pltpu.sync_copy(idx_hbm.at[pl.ds(chunk_base, CHUNKS_PER_WORKER)], idx_v)

    def chunk_body(c, carry):
        pltpu.async_copy(lut_hbm.at[idx_v.at[c]], rows_v, sem).wait()
        phase = (c * CHUNK) % SEQ_LEN

        def row_body(r, rc):
            pr = phase + r
            for j in range(D_MODEL // LANES):
                sl = pl.ds(j * LANES, LANES)
                rows_v[r, sl] = rows_v[r, sl] * SCALE + pe_v[pr, sl]
            return rc

        lax.fori_loop(0, CHUNK, row_body, 0)
        pltpu.sync_copy(
            rows_v, out_hbm.at[pl.ds((chunk_base + c) * CHUNK, CHUNK)]
        )
        return carry

    lax.fori_loop(0, CHUNKS_PER_WORKER, chunk_body, 0)


@jax.jit
def _sc_call(lut, idx2d, pe):
    mesh = plsc.VectorSubcoreMesh(core_axis_name="c", subcore_axis_name="s")
    return pl.kernel(
        _sc_body,
        out_type=jax.ShapeDtypeStruct((TOTAL_ROWS, D_MODEL), jnp.float32),
        mesh=mesh,
        scratch_types=[
            pltpu.VMEM((CHUNKS_PER_WORKER, CHUNK), jnp.int32),
            pltpu.VMEM((PE_ROWS, D_MODEL), jnp.float32),
            pltpu.VMEM((CHUNK, D_MODEL), jnp.float32),
            pltpu.SemaphoreType.DMA,
        ],
    )(lut, idx2d, pe)


def kernel(x, lut):
    idx2d = jnp.reshape(x.astype(jnp.int32), (CHUNKS_TOTAL, CHUNK))
    out = _sc_call(lut, idx2d, jnp.asarray(_PE2))
    return jnp.reshape(out, (x.shape[0], SEQ_LEN, D_MODEL))


# speedup vs baseline: 1.3939x; 1.3939x over previous
import math

import numpy as np
import jax
import jax.numpy as jnp
from jax import lax
from jax.experimental import pallas as pl
from jax.experimental.pallas import tpu as pltpu
from jax.experimental.pallas import tpu_sc as plsc

D_MODEL = 128
SEQ_LEN = 200
SCALE = math.sqrt(float(D_MODEL))
CHUNK = 80
NBUF = 4
PREF = 2
NUM_WORKERS = 32
TOTAL_ROWS = 4096 * SEQ_LEN
CHUNKS_TOTAL = TOTAL_ROWS // CHUNK
CHUNKS_PER_WORKER = CHUNKS_TOTAL // NUM_WORKERS
PE_ROWS = 240
LANES = 16


def _make_pe_const() -> np.ndarray:
    position = np.arange(0, SEQ_LEN, dtype=np.float32)[:, None]
    div_term = np.exp(
        np.arange(0, D_MODEL, 2, dtype=np.float32) * -(math.log(10000.0) / D_MODEL)
    )
    pe = np.zeros((SEQ_LEN, D_MODEL), dtype=np.float32)
    pe[:, 0::2] = np.sin(position * div_term)
    pe[:, 1::2] = np.cos(position * div_term)
    return np.concatenate([pe, pe], axis=0)[:PE_ROWS]


_PE2 = _make_pe_const()


def _sc_body(lut_hbm, idx_hbm, pe_hbm, out_hbm, idx_v, pe_v, rows_v, gsems, osems):
    core = lax.axis_index("c")
    sub = lax.axis_index("s")
    wid = sub * 2 + core
    chunk_base = wid * CHUNKS_PER_WORKER

    pltpu.sync_copy(pe_hbm, pe_v)
    pltpu.sync_copy(idx_hbm.at[pl.ds(chunk_base, CHUNKS_PER_WORKER)], idx_v)

    def start_gather(c, b):
        pltpu.async_copy(lut_hbm.at[idx_v.at[c]], rows_v.at[b], gsems[b])

    def wait_gather(b):
        pltpu.make_async_copy(
            out_hbm.at[pl.ds(0, CHUNK)], rows_v.at[b], gsems[b]
        ).wait()

    def start_store(c, b):
        pltpu.async_copy(
            rows_v.at[b], out_hbm.at[pl.ds((chunk_base + c) * CHUNK, CHUNK)],
            osems[b],
        )

    def wait_store(b):
        pltpu.make_async_copy(
            rows_v.at[b], out_hbm.at[pl.ds(0, CHUNK)], osems[b]
        ).wait()

    for b in range(PREF):
        start_gather(b, b % NBUF)

    def group_body(g, carry):
        for b in range(NBUF):
            c = g * NBUF + b
            wait_gather(b)

            def row_body(r, rc):
                pr = (c * CHUNK) % SEQ_LEN + r
                for j in range(D_MODEL // LANES):
                    sl = pl.ds(j * LANES, LANES)
                    rows_v[b, r, sl] = rows_v[b, r, sl] * SCALE + pe_v[pr, sl]
                return rc

            lax.fori_loop(0, CHUNK, row_body, 0)
            start_store(c, b)

            pc = c + PREF
            pb = (b + PREF) % NBUF

            @pl.when(pc < CHUNKS_PER_WORKER)
            def _prefetch():
                @pl.when(pc - NBUF >= 0)
                def _drain():
                    wait_store(pb)

                start_gather(pc, pb)

        return carry

    lax.fori_loop(0, CHUNKS_PER_WORKER // NBUF, group_body, 0)

    for b in range(NBUF):
        wait_store(b)


@jax.jit
def _sc_call(lut, idx2d, pe):
    mesh = plsc.VectorSubcoreMesh(core_axis_name="c", subcore_axis_name="s")
    return pl.kernel(
        _sc_body,
        out_type=jax.ShapeDtypeStruct((TOTAL_ROWS, D_MODEL), jnp.float32),
        mesh=mesh,
        scratch_types=[
            pltpu.VMEM((CHUNKS_PER_WORKER, CHUNK), jnp.int32),
            pltpu.VMEM((PE_ROWS, D_MODEL), jnp.float32),
            pltpu.VMEM((NBUF, CHUNK, D_MODEL), jnp.float32),
            [pltpu.SemaphoreType.DMA] * NBUF,
            [pltpu.SemaphoreType.DMA] * NBUF,
        ],
    )(lut, idx2d, pe)


def kernel(x, lut):
    idx2d = jnp.reshape(x.astype(jnp.int32), (CHUNKS_TOTAL, CHUNK))
    out = _sc_call(lut, idx2d, jnp.asarray(_PE2))
    return jnp.reshape(out, (x.shape[0], SEQ_LEN, D_MODEL))
